# baseline (speedup 1.0000x reference)
import functools

import jax
import jax.numpy as jnp
from jax import lax
from jax.experimental import pallas as pl


def _search_kernel(N, D, NPc, K, xr_ref, w_ref, z_ref, t_ref):
  acc = jnp.zeros((8, NPc), jnp.float32)
  for d in range(D):
    acc = acc + xr_ref[pl.ds(d * 8, 8), :] * w_ref[0, d]
  z_ref[...] = acc
  bits = lax.bitcast_convert_type(acc, jnp.int32)
  flip = jnp.where(bits < 0, jnp.int32(-1), jnp.int32(-2147483648))
  key = (bits ^ flip).astype(jnp.uint32)
  srow = lax.broadcasted_iota(jnp.int32, (8, NPc), 0)
  scol = lax.broadcasted_iota(jnp.int32, (8, NPc), 1)
  key = jnp.where(srow * NPc + scol < N, key, jnp.uint32(0))
  def step(i, t):
    cand = t | lax.shift_left(jnp.uint32(1),
                              jnp.uint32(31) - i.astype(jnp.uint32))
    cnt = jnp.sum(jnp.where(key >= cand, jnp.int32(1), jnp.int32(0)))
    return jnp.where(cnt >= K, cand, t)
  t_ref[...] = jnp.broadcast_to(lax.fori_loop(0, 32, step, jnp.uint32(0)),
                                (1, 1))


def _key_of(z):
  bits = lax.bitcast_convert_type(z, jnp.int32)
  flip = jnp.where(bits < 0, jnp.int32(-1), jnp.int32(-2147483648))
  return (bits ^ flip).astype(jnp.uint32)


def _pool_stage(x, w, K, NPAD):
  N, D = x.shape
  NPc = NPAD // 8
  wn = (w / jnp.linalg.norm(w)).astype(jnp.float32)
  wp = jnp.zeros((1, 128), jnp.float32).at[0, :D].set(wn)
  xr = (jnp.zeros((D, NPAD), jnp.float32).at[:, :N].set(x.T)
        .reshape(D * 8, NPc))
  z8, t = pl.pallas_call(
      functools.partial(_search_kernel, N, D, NPc, K),
      out_shape=[jax.ShapeDtypeStruct((8, NPc), jnp.float32),
                 jax.ShapeDtypeStruct((1, 1), jnp.uint32)],
  )(xr, wp)
  z = z8.reshape(NPAD)[:N]
  key = _key_of(z)
  thr = t[0, 0]
  mgt = key > thr
  g = jnp.sum(mgt.astype(jnp.int32))
  idx_gt = jnp.nonzero(mgt, size=K, fill_value=0)[0]
  idx_eq = jnp.nonzero(key == thr, size=K, fill_value=0)[0]
  pos = jnp.arange(K, dtype=jnp.int32)
  sel = jnp.where(pos < g,
                  idx_gt[jnp.minimum(pos, K - 1)],
                  idx_eq[jnp.clip(pos - g, 0, K - 1)])
  return sel, z


def _scale_kernel(x_ref, z_ref, o_ref):
  o_ref[...] = x_ref[...] * jnp.tanh(z_ref[...])


def _scale(x, z):
  return pl.pallas_call(
      _scale_kernel,
      out_shape=jax.ShapeDtypeStruct(x.shape, jnp.float32),
  )(x, z[:, None])


def _scale_linear_kernel(x_ref, z_ref, w_ref, b_ref, o_ref):
  xs = x_ref[...] * jnp.tanh(z_ref[...])
  o_ref[...] = jnp.dot(xs, w_ref[...].T,
                       preferred_element_type=jnp.float32) + b_ref[...]


def _scale_linear(x, z, W, b):
  return pl.pallas_call(
      _scale_linear_kernel,
      out_shape=jax.ShapeDtypeStruct((x.shape[0], W.shape[0]), jnp.float32),
  )(x, z[:, None], W, b[None, :])


def _matmul_kernel(x_ref, w_ref, b_ref, o_ref):
  o_ref[...] = jnp.dot(x_ref[...], w_ref[...].T,
                       preferred_element_type=jnp.float32) + b_ref[...]


def _linear(x, W, b):
  return pl.pallas_call(
      _matmul_kernel,
      out_shape=jax.ShapeDtypeStruct((x.shape[0], W.shape[0]), jnp.float32),
  )(x, W, b[None, :])


def kernel(x, edge_index, batch, pool1_w, lin1_W, lin1_b, pool2_w, pool3_w,
           lin2_W, lin2_b):
  sel1, z1 = _pool_stage(x, pool1_w, K=1000, NPAD=100096)
  x1 = _scale_linear(x[sel1], z1[sel1], lin1_W, lin1_b)
  sel2, z2 = _pool_stage(x1, pool2_w, K=100, NPAD=1024)
  x2 = _scale(x1[sel2], z2[sel2])
  sel3, z3 = _pool_stage(x2, pool3_w, K=10, NPAD=1024)
  z3s = z3[sel3]
  order = jnp.argsort(-z3s)
  x3 = _scale(x2[sel3], z3s)[order]
  return _linear(x3, lin2_W, lin2_b)
